# baseline (device time: 46345 ns/iter reference)
import jax
import jax.numpy as jnp
from jax import lax
from jax.experimental import pallas as pl
from jax.experimental.pallas import tpu as pltpu

N_DEV = 4
N_HOP = N_DEV - 1
S = 2


def kernel(x, w_mat):
    k, n = w_mat.shape
    m = x.shape[0]
    m_per = m // N_DEV
    n_half = n // 2
    sub = n_half // S

    def body(
        x_ref, w_ref, out_ref,
        send_a, send_b, recv_a, recv_b,
        ss_a, rs_a, ss_b, rs_b,
    ):
        my = lax.axis_index("i")
        left = lax.rem(my + (N_DEV - 1), N_DEV)
        right = lax.rem(my + 1, N_DEV)

        barrier_sem = pltpu.get_barrier_semaphore()
        for nbr in (left, right):
            pl.semaphore_signal(
                barrier_sem, inc=1,
                device_id=(nbr,), device_id_type=pl.DeviceIdType.MESH,
            )
        pl.semaphore_wait(barrier_sem, 2)

        w = w_ref[...].astype(jnp.bfloat16)

        def partial(c, lo=0, width=None):
            xs = x_ref[pl.ds(c * m_per, m_per), :].astype(jnp.bfloat16)
            ws = w if width is None else w[:, lo:lo + width]
            return jnp.dot(xs, ws, preferred_element_type=jnp.float32)

        c_dm1 = lax.rem(my + N_DEV - 1, N_DEV)
        c_dp1 = lax.rem(my + 1, N_DEV)
        c_dp2 = lax.rem(my + 2, N_DEV)

        def mk(h, s, tgt, src, dst, ssem, rsem):
            return pltpu.make_async_remote_copy(
                src_ref=src.at[h, s], dst_ref=dst.at[h, s],
                send_sem=ssem.at[h, s], recv_sem=rsem.at[h, s],
                device_id=(tgt,), device_id_type=pl.DeviceIdType.MESH,
            )

        def mk_a(h, s):
            return mk(h, s, right, send_a, recv_a, ss_a, rs_a)

        def mk_b(h, s):
            return mk(h, s, left, send_b, recv_b, ss_b, rs_b)

        def fwd(h, s, add_a, add_b):
            mk_a(h, s).wait_recv()
            send_a[h + 1, s] = (
                recv_a[h, s].astype(jnp.float32) + add_a
            ).astype(jnp.bfloat16)
            mk_a(h + 1, s).start()
            mk_b(h, s).wait_recv()
            send_b[h + 1, s] = (
                recv_b[h, s].astype(jnp.float32) + add_b
            ).astype(jnp.bfloat16)
            mk_b(h + 1, s).start()

        for s in range(S):
            send_a[0, s] = partial(c_dm1, s * sub, sub).astype(jnp.bfloat16)
            mk_a(0, s).start()
            send_b[0, s] = partial(
                c_dp1, n_half + s * sub, sub
            ).astype(jnp.bfloat16)
            mk_b(0, s).start()

        pc = partial(c_dp2)

        fwd(0, 0, pc[:, :sub], pc[:, n_half:n_half + sub])

        p1a = partial(c_dp1, 0, n_half)
        p1b = partial(c_dm1, n_half, n_half)

        fwd(0, 1, pc[:, sub:2 * sub], pc[:, n_half + sub:n_half + 2 * sub])

        pd = partial(my)

        fwd(1, 0, p1a[:, :sub], p1b[:, :sub])
        fwd(1, 1, p1a[:, sub:2 * sub], p1b[:, sub:2 * sub])

        for s in range(S):
            mk_a(N_HOP - 1, s).wait_recv()
            out_ref[:, s * sub:(s + 1) * sub] = jnp.maximum(
                recv_a[N_HOP - 1, s].astype(jnp.float32)
                + pd[:, s * sub:(s + 1) * sub], 0.0,
            )
            mk_b(N_HOP - 1, s).wait_recv()
            lo = n_half + s * sub
            out_ref[:, lo:lo + sub] = jnp.maximum(
                recv_b[N_HOP - 1, s].astype(jnp.float32)
                + pd[:, lo:lo + sub], 0.0,
            )

        for h in range(N_HOP):
            for s in range(S):
                mk_a(h, s).wait_send()
                mk_b(h, s).wait_send()

    comm = pltpu.VMEM((N_HOP, S, m_per, sub), jnp.bfloat16)
    sems = pltpu.SemaphoreType.DMA((N_HOP, S))
    return pl.pallas_call(
        body,
        out_shape=jax.ShapeDtypeStruct((m_per, n), jnp.float32),
        in_specs=[
            pl.BlockSpec(memory_space=pltpu.VMEM),
            pl.BlockSpec(memory_space=pltpu.VMEM),
        ],
        out_specs=pl.BlockSpec(memory_space=pltpu.VMEM),
        scratch_shapes=[comm, comm, comm, comm, sems, sems, sems, sems],
        compiler_params=pltpu.CompilerParams(collective_id=0),
    )(x, w_mat)


# device time: 45714 ns/iter; 1.0138x vs baseline; 1.0138x over previous
import jax
import jax.numpy as jnp
from jax import lax
from jax.experimental import pallas as pl
from jax.experimental.pallas import tpu as pltpu

N_DEV = 4

SEED_A, DIR_B, RELAY_A, SEED_B, DIR_A, RELAY_B = range(6)
_TO_RIGHT = {SEED_A, DIR_B, RELAY_A}


def kernel(x, w_mat):
    k, n = w_mat.shape
    m = x.shape[0]
    m_per = m // N_DEV
    n_half = n // 2

    def body(x_ref, w_ref, out_ref, sbuf, rbuf, ss, rs):
        my = lax.axis_index("i")
        left = lax.rem(my + (N_DEV - 1), N_DEV)
        right = lax.rem(my + 1, N_DEV)

        barrier_sem = pltpu.get_barrier_semaphore()
        for nbr in (left, right):
            pl.semaphore_signal(
                barrier_sem, inc=1,
                device_id=(nbr,), device_id_type=pl.DeviceIdType.MESH,
            )
        pl.semaphore_wait(barrier_sem, 2)

        w = w_ref[...].astype(jnp.bfloat16)

        def partial(c, lo):
            xs = x_ref[pl.ds(c * m_per, m_per), :].astype(jnp.bfloat16)
            return jnp.dot(
                xs, w[:, lo:lo + n_half], preferred_element_type=jnp.float32
            )

        c_dm1 = lax.rem(my + N_DEV - 1, N_DEV)
        c_dp1 = lax.rem(my + 1, N_DEV)
        c_dp2 = lax.rem(my + 2, N_DEV)

        def mk(slot):
            return pltpu.make_async_remote_copy(
                src_ref=sbuf.at[slot], dst_ref=rbuf.at[slot],
                send_sem=ss.at[slot], recv_sem=rs.at[slot],
                device_id=(right if slot in _TO_RIGHT else left,),
                device_id_type=pl.DeviceIdType.MESH,
            )

        sbuf[SEED_A] = partial(c_dp2, 0).astype(jnp.bfloat16)
        mk(SEED_A).start()
        sbuf[SEED_B] = partial(c_dp2, n_half).astype(jnp.bfloat16)
        mk(SEED_B).start()
        sbuf[DIR_B] = partial(c_dp1, n_half).astype(jnp.bfloat16)
        mk(DIR_B).start()
        sbuf[DIR_A] = partial(c_dm1, 0).astype(jnp.bfloat16)
        mk(DIR_A).start()

        p_relay_a = partial(c_dp1, 0)
        p_relay_b = partial(c_dm1, n_half)
        pd_a = partial(my, 0)
        pd_b = partial(my, n_half)

        mk(SEED_A).wait_recv()
        sbuf[RELAY_A] = (
            rbuf[SEED_A].astype(jnp.float32) + p_relay_a
        ).astype(jnp.bfloat16)
        mk(RELAY_A).start()
        mk(SEED_B).wait_recv()
        sbuf[RELAY_B] = (
            rbuf[SEED_B].astype(jnp.float32) + p_relay_b
        ).astype(jnp.bfloat16)
        mk(RELAY_B).start()

        mk(DIR_A).wait_recv()
        mk(RELAY_A).wait_recv()
        out_ref[:, :n_half] = jnp.maximum(
            rbuf[RELAY_A].astype(jnp.float32)
            + rbuf[DIR_A].astype(jnp.float32) + pd_a, 0.0,
        )
        mk(DIR_B).wait_recv()
        mk(RELAY_B).wait_recv()
        out_ref[:, n_half:] = jnp.maximum(
            rbuf[RELAY_B].astype(jnp.float32)
            + rbuf[DIR_B].astype(jnp.float32) + pd_b, 0.0,
        )

        for slot in range(6):
            mk(slot).wait_send()

    comm = pltpu.VMEM((6, m_per, n_half), jnp.bfloat16)
    sems = pltpu.SemaphoreType.DMA((6,))
    return pl.pallas_call(
        body,
        out_shape=jax.ShapeDtypeStruct((m_per, n), jnp.float32),
        in_specs=[
            pl.BlockSpec(memory_space=pltpu.VMEM),
            pl.BlockSpec(memory_space=pltpu.VMEM),
        ],
        out_specs=pl.BlockSpec(memory_space=pltpu.VMEM),
        scratch_shapes=[comm, comm, sems, sems],
        compiler_params=pltpu.CompilerParams(collective_id=0),
    )(x, w_mat)
